# baseline (speedup 1.0000x reference)
import functools

import jax
import jax.numpy as jnp
from jax import lax
from jax.experimental import pallas as pl
from jax.experimental.pallas import tpu as pltpu
from jax.experimental.pallas import tpu_sc as plsc

VOCAB = 100000
D_HALF = 64
D_MODEL = 128
N_TOKENS = 4096 * 50

NW = 32
BPW = N_TOKENS // NW
IDXW = 128
CHUNK = 256
IDX_PER_CHUNK = CHUNK // IDXW
NCHUNK = BPW // CHUNK


def _table_transform_tc(emb_table, W_comb):
    BR = 4000
    grid = (VOCAB // BR,)

    def body(t_ref, w_ref, o_ref):
        o_ref[...] = jnp.dot(t_ref[...], w_ref[:D_HALF, :],
                             preferred_element_type=jnp.float32)

    return pl.pallas_call(
        body,
        grid=grid,
        in_specs=[
            pl.BlockSpec((BR, D_HALF), lambda i: (i, 0)),
            pl.BlockSpec((D_MODEL, D_MODEL), lambda i: (0, 0)),
        ],
        out_specs=pl.BlockSpec((BR, D_MODEL), lambda i: (i, 0)),
        out_shape=jax.ShapeDtypeStruct((VOCAB, D_MODEL), jnp.float32),
    )(emb_table, W_comb)


def _gather_sc(table_t, ids_flat):
    mesh = plsc.VectorSubcoreMesh(core_axis_name="c", subcore_axis_name="s")

    @functools.partial(
        pl.kernel,
        mesh=mesh,
        compiler_params=pltpu.CompilerParams(use_tc_tiling_on_sc=True),
        out_type=jax.ShapeDtypeStruct((N_TOKENS, D_MODEL), jnp.float32),
        scratch_types=[
            pltpu.VMEM((BPW,), jnp.int32),
            pltpu.VMEM((CHUNK, D_MODEL), jnp.float32),
            pltpu.VMEM((CHUNK, D_MODEL), jnp.float32),
            pltpu.SemaphoreType.DMA,
            pltpu.SemaphoreType.DMA,
            pltpu.SemaphoreType.DMA,
        ],
    )
    def k(table_hbm, ids_hbm, out_hbm, idx_v, buf0, buf1, gsem, osem0, osem1):
        wid = lax.axis_index("s") * 2 + lax.axis_index("c")
        base = wid * BPW
        pltpu.sync_copy(ids_hbm.at[pl.ds(base, BPW)], idx_v)
        bufs = (buf0, buf1)
        osems = (osem0, osem1)

        def issue(c, buf):
            hs = []
            for j in range(IDX_PER_CHUNK):
                hs.append(pltpu.async_copy(
                    table_hbm.at[idx_v.at[pl.ds((c * IDX_PER_CHUNK + j) * IDXW, IDXW)]],
                    buf.at[pl.ds(j * IDXW, IDXW)],
                    gsem))
            return hs

        pending = issue(0, bufs[0])
        out_h = [None, None]
        for c in range(NCHUNK):
            b = c & 1
            for h in pending:
                h.wait()
            if c + 1 < NCHUNK:
                if out_h[1 - b] is not None:
                    out_h[1 - b].wait()
                pending = issue(c + 1, bufs[1 - b])
            out_h[b] = pltpu.async_copy(
                bufs[b], out_hbm.at[pl.ds(base + c * CHUNK, CHUNK)], osems[b])
        out_h[(NCHUNK - 1) & 1].wait()

    return k(table_t, ids_flat)


def _dense_tc(gathered, stats2d, W_stat, b_stat, W_comb, b_comb):
    ROWS = 2048
    grid = (N_TOKENS // ROWS,)

    def body(g_ref, st_ref, ws_ref, bs_ref, wc_ref, bc_ref, out_ref):
        s = jnp.dot(st_ref[...], ws_ref[...],
                    preferred_element_type=jnp.float32) + bs_ref[...]
        s = 0.5 * s * (1.0 + lax.erf(s * 0.7071067811865476))
        bot = jnp.dot(s, wc_ref[D_HALF:, :],
                      preferred_element_type=jnp.float32)
        out_ref[...] = g_ref[...] + bot + bc_ref[...]

    return pl.pallas_call(
        body,
        grid=grid,
        in_specs=[
            pl.BlockSpec((ROWS, D_MODEL), lambda i: (i, 0)),
            pl.BlockSpec((ROWS, 10), lambda i: (i, 0)),
            pl.BlockSpec((10, D_HALF), lambda i: (0, 0)),
            pl.BlockSpec((1, D_HALF), lambda i: (0, 0)),
            pl.BlockSpec((D_MODEL, D_MODEL), lambda i: (0, 0)),
            pl.BlockSpec((1, D_MODEL), lambda i: (0, 0)),
        ],
        out_specs=pl.BlockSpec((ROWS, D_MODEL), lambda i: (i, 0)),
        out_shape=jax.ShapeDtypeStruct((N_TOKENS, D_MODEL), jnp.float32),
    )(gathered, stats2d, W_stat, b_stat.reshape(1, D_HALF),
      W_comb, b_comb.reshape(1, D_MODEL))


def kernel(card_ids, card_stats, emb_table, W_stat, b_stat, W_comb, b_comb):
    B, L = card_ids.shape
    ids_flat = card_ids.reshape(N_TOKENS).astype(jnp.int32)
    table_t = _table_transform_tc(emb_table, W_comb)
    gathered = _gather_sc(table_t, ids_flat)
    stats2d = card_stats.reshape(N_TOKENS, 10)
    out = _dense_tc(gathered, stats2d, W_stat, b_stat, W_comb, b_comb)
    return out.reshape(B, L, D_MODEL)


# speedup vs baseline: 1.7950x; 1.7950x over previous
import functools

import jax
import jax.numpy as jnp
from jax import lax
from jax.experimental import pallas as pl
from jax.experimental.pallas import tpu as pltpu
from jax.experimental.pallas import tpu_sc as plsc

VOCAB = 100000
D_HALF = 64
D_MODEL = 128
N_TOKENS = 4096 * 50

NW = 32
BPW = N_TOKENS // NW
IDXW = 128
CHUNK = 256
IDX_PER_CHUNK = CHUNK // IDXW
NCHUNK = BPW // CHUNK


def _table_transform_tc(emb_table, W_comb):
    BR = 4000
    grid = (VOCAB // BR,)

    def body(t_ref, w_ref, o_ref):
        o_ref[...] = jnp.dot(t_ref[...], w_ref[:D_HALF, :],
                             preferred_element_type=jnp.float32)

    return pl.pallas_call(
        body,
        grid=grid,
        in_specs=[
            pl.BlockSpec((BR, D_HALF), lambda i: (i, 0)),
            pl.BlockSpec((D_MODEL, D_MODEL), lambda i: (0, 0)),
        ],
        out_specs=pl.BlockSpec((BR, D_MODEL), lambda i: (i, 0)),
        out_shape=jax.ShapeDtypeStruct((VOCAB, D_MODEL), jnp.float32),
    )(emb_table, W_comb)


def _gather_sc(table_t, ids_flat):
    mesh = plsc.VectorSubcoreMesh(core_axis_name="c", subcore_axis_name="s")

    @functools.partial(
        pl.kernel,
        mesh=mesh,
        compiler_params=pltpu.CompilerParams(use_tc_tiling_on_sc=True),
        out_type=jax.ShapeDtypeStruct((N_TOKENS, D_MODEL), jnp.float32),
        scratch_types=[
            pltpu.VMEM((BPW,), jnp.int32),
            pltpu.VMEM((CHUNK, D_MODEL), jnp.float32),
            pltpu.VMEM((CHUNK, D_MODEL), jnp.float32),
            pltpu.SemaphoreType.DMA,
            pltpu.SemaphoreType.DMA,
            pltpu.SemaphoreType.DMA,
        ],
    )
    def k(table_hbm, ids_hbm, out_hbm, idx_v, buf0, buf1, gsem, osem0, osem1):
        wid = lax.axis_index("s") * 2 + lax.axis_index("c")
        base = wid * BPW
        pltpu.sync_copy(ids_hbm.at[pl.ds(base, BPW)], idx_v)
        bufs = (buf0, buf1)
        osems = (osem0, osem1)

        def issue(c, buf):
            hs = []
            for j in range(IDX_PER_CHUNK):
                hs.append(pltpu.async_copy(
                    table_hbm.at[idx_v.at[pl.ds((c * IDX_PER_CHUNK + j) * IDXW, IDXW)]],
                    buf.at[pl.ds(j * IDXW, IDXW)],
                    gsem))
            return hs

        pending = issue(0, bufs[0])
        out_h = [None, None]
        for c in range(NCHUNK):
            b = c & 1
            for h in pending:
                h.wait()
            if c + 1 < NCHUNK:
                if out_h[1 - b] is not None:
                    out_h[1 - b].wait()
                pending = issue(c + 1, bufs[1 - b])
            out_h[b] = pltpu.async_copy(
                bufs[b], out_hbm.at[pl.ds(base + c * CHUNK, CHUNK)], osems[b])
        out_h[(NCHUNK - 1) & 1].wait()

    return k(table_t, ids_flat)


def _dense_tc(gathered, stats2d, W_stat, b_stat, W_comb, b_comb):
    ROWS = 2048
    grid = (N_TOKENS // ROWS,)

    def body(g_ref, st_ref, ws_ref, bs_ref, wc_ref, bc_ref, out_ref):
        s = jnp.dot(st_ref[...], ws_ref[...],
                    preferred_element_type=jnp.float32) + bs_ref[...]
        s = 0.5 * s * (1.0 + lax.erf(s * 0.7071067811865476))
        bot = jnp.dot(s, wc_ref[D_HALF:, :],
                      preferred_element_type=jnp.float32)
        out_ref[...] = g_ref[...] + bot + bc_ref[...]

    return pl.pallas_call(
        body,
        grid=grid,
        in_specs=[
            pl.BlockSpec((ROWS, D_MODEL), lambda i: (i, 0)),
            pl.BlockSpec((ROWS, 10), lambda i: (i, 0)),
            pl.BlockSpec((10, D_HALF), lambda i: (0, 0)),
            pl.BlockSpec((1, D_HALF), lambda i: (0, 0)),
            pl.BlockSpec((D_MODEL, D_MODEL), lambda i: (0, 0)),
            pl.BlockSpec((1, D_MODEL), lambda i: (0, 0)),
        ],
        out_specs=pl.BlockSpec((ROWS, D_MODEL), lambda i: (i, 0)),
        out_shape=jax.ShapeDtypeStruct((N_TOKENS, D_MODEL), jnp.float32),
    )(gathered, stats2d, W_stat, b_stat.reshape(1, D_HALF),
      W_comb, b_comb.reshape(1, D_MODEL))


def kernel(card_ids, card_stats, emb_table, W_stat, b_stat, W_comb, b_comb):
    B, L = card_ids.shape
    ids_flat = card_ids.T.reshape(N_TOKENS).astype(jnp.int32)
    table_t = _table_transform_tc(emb_table, W_comb)
    gathered = _gather_sc(table_t, ids_flat)
    stats2d = card_stats.transpose(1, 0, 2).reshape(N_TOKENS, 10)
    out = _dense_tc(gathered, stats2d, W_stat, b_stat, W_comb, b_comb)
    return out.reshape(L, B, D_MODEL).transpose(1, 0, 2)


# speedup vs baseline: 1.8503x; 1.0308x over previous
import functools

import jax
import jax.numpy as jnp
from jax import lax
from jax.experimental import pallas as pl
from jax.experimental.pallas import tpu as pltpu
from jax.experimental.pallas import tpu_sc as plsc

VOCAB = 100000
D_HALF = 64
D_MODEL = 128
N_TOKENS = 4096 * 50
F_PAD = 16
TOK_PER_ROW = 128 // F_PAD

NW = 32
BPW = N_TOKENS // NW
IDXW = 128
CHUNK = 256
IDX_PER_CHUNK = CHUNK // IDXW
NCHUNK = BPW // CHUNK


def _table_transform_tc(emb_table_T, W_comb):
    BR = 4096
    grid = ((VOCAB + BR - 1) // BR,)

    def body(t_ref, w_ref, o_ref):
        o_ref[...] = jax.lax.dot_general(
            t_ref[...], w_ref[:D_HALF, :],
            dimension_numbers=(((0,), (0,)), ((), ())),
            preferred_element_type=jnp.float32)

    return pl.pallas_call(
        body,
        grid=grid,
        in_specs=[
            pl.BlockSpec((D_HALF, BR), lambda i: (0, i)),
            pl.BlockSpec((D_MODEL, D_MODEL), lambda i: (0, 0)),
        ],
        out_specs=pl.BlockSpec((BR, D_MODEL), lambda i: (i, 0)),
        out_shape=jax.ShapeDtypeStruct((VOCAB, D_MODEL), jnp.float32),
    )(emb_table_T, W_comb)


def _gather_sc(table_t, ids_flat):
    mesh = plsc.VectorSubcoreMesh(core_axis_name="c", subcore_axis_name="s")

    @functools.partial(
        pl.kernel,
        mesh=mesh,
        out_type=jax.ShapeDtypeStruct((N_TOKENS, D_MODEL), jnp.float32),
        scratch_types=[
            pltpu.VMEM((BPW,), jnp.int32),
            pltpu.VMEM((CHUNK, D_MODEL), jnp.float32),
            pltpu.VMEM((CHUNK, D_MODEL), jnp.float32),
            pltpu.SemaphoreType.DMA,
            pltpu.SemaphoreType.DMA,
            pltpu.SemaphoreType.DMA,
        ],
    )
    def k(table_hbm, ids_hbm, out_hbm, idx_v, buf0, buf1, gsem, osem0, osem1):
        wid = lax.axis_index("s") * 2 + lax.axis_index("c")
        base = wid * BPW
        pltpu.sync_copy(ids_hbm.at[pl.ds(base, BPW)], idx_v)
        bufs = (buf0, buf1)
        osems = (osem0, osem1)

        def issue(c, buf):
            hs = []
            for j in range(IDX_PER_CHUNK):
                hs.append(pltpu.async_copy(
                    table_hbm.at[idx_v.at[pl.ds((c * IDX_PER_CHUNK + j) * IDXW, IDXW)]],
                    buf.at[pl.ds(j * IDXW, IDXW)],
                    gsem))
            return hs

        pending = issue(0, bufs[0])
        out_h = [None, None]
        for c in range(NCHUNK):
            b = c & 1
            for h in pending:
                h.wait()
            if c + 1 < NCHUNK:
                if out_h[1 - b] is not None:
                    out_h[1 - b].wait()
                pending = issue(c + 1, bufs[1 - b])
            out_h[b] = pltpu.async_copy(
                bufs[b], out_hbm.at[pl.ds(base + c * CHUNK, CHUNK)], osems[b])
        out_h[(NCHUNK - 1) & 1].wait()

    return k(table_t, ids_flat)


def _dense_tc(gathered, stats_pack, W_big, b_big, W_bot_diag, b_comb):
    ROWS = 2048
    PROWS = ROWS // TOK_PER_ROW
    grid = (N_TOKENS // ROWS,)

    def body(g_ref, sp_ref, wb_ref, bb_ref, wbd_ref, bc_ref, out_ref):
        s = jnp.dot(sp_ref[...], wb_ref[...],
                    preferred_element_type=jnp.float32) + bb_ref[...]
        s = 0.5 * s * (1.0 + lax.erf(s * 0.7071067811865476))
        o = jnp.dot(s, wbd_ref[...], preferred_element_type=jnp.float32)
        for g in range(TOK_PER_ROW):
            sl = pl.ds(g * PROWS, PROWS)
            out_ref[sl, :] = (g_ref[sl, :]
                              + o[:, g * D_MODEL:(g + 1) * D_MODEL]
                              + bc_ref[...])

    return pl.pallas_call(
        body,
        grid=grid,
        in_specs=[
            pl.BlockSpec((ROWS, D_MODEL), lambda i: (i, 0)),
            pl.BlockSpec((PROWS, D_MODEL), lambda i: (i, 0)),
            pl.BlockSpec((D_MODEL, 8 * D_HALF), lambda i: (0, 0)),
            pl.BlockSpec((1, 8 * D_HALF), lambda i: (0, 0)),
            pl.BlockSpec((8 * D_HALF, 8 * D_MODEL), lambda i: (0, 0)),
            pl.BlockSpec((1, D_MODEL), lambda i: (0, 0)),
        ],
        out_specs=pl.BlockSpec((ROWS, D_MODEL), lambda i: (i, 0)),
        out_shape=jax.ShapeDtypeStruct((N_TOKENS, D_MODEL), jnp.float32),
    )(gathered, stats_pack, W_big, b_big.reshape(1, 8 * D_HALF),
      W_bot_diag, b_comb.reshape(1, D_MODEL))


def kernel(card_ids, card_stats, emb_table, W_stat, b_stat, W_comb, b_comb):
    B, L = card_ids.shape
    ids_flat = card_ids.T.reshape(N_TOKENS).astype(jnp.int32)
    table_t = _table_transform_tc(emb_table.T, W_comb)
    gathered = _gather_sc(table_t, ids_flat)

    stats_lm = card_stats.transpose(1, 0, 2)
    stats_pad = jnp.pad(stats_lm, ((0, 0), (0, 0), (0, F_PAD - 10)))
    PROWS = 2048 // TOK_PER_ROW
    stats_pack = (stats_pad.reshape(N_TOKENS // 2048, TOK_PER_ROW, PROWS, F_PAD)
                  .transpose(0, 2, 1, 3)
                  .reshape(N_TOKENS // TOK_PER_ROW, 128))

    W16 = jnp.pad(W_stat, ((0, F_PAD - 10), (0, 0)))
    eye8 = jnp.eye(TOK_PER_ROW, dtype=jnp.float32)
    W_big = jnp.einsum("gh,fo->gfho", eye8, W16).reshape(128, 8 * D_HALF)
    b_big = jnp.tile(b_stat, TOK_PER_ROW)
    W_bot_diag = jnp.einsum("gh,fo->gfho", eye8,
                            W_comb[D_HALF:, :]).reshape(8 * D_HALF, 8 * D_MODEL)

    out = _dense_tc(gathered, stats_pack, W_big, b_big, W_bot_diag, b_comb)
    return out.reshape(L, B, D_MODEL).transpose(1, 0, 2)


# speedup vs baseline: 1.9252x; 1.0405x over previous
import functools

import jax
import jax.numpy as jnp
from jax import lax
from jax.experimental import pallas as pl
from jax.experimental.pallas import tpu as pltpu
from jax.experimental.pallas import tpu_sc as plsc

VOCAB = 100000
D_HALF = 64
D_MODEL = 128
N_TOKENS = 4096 * 50
N_HALF = N_TOKENS // 2
F_PAD = 16
TOK_PER_ROW = 128 // F_PAD
ROWS = 2048
PROWS = ROWS // TOK_PER_ROW

NW = 32
IDXW = 128


def _table_transform_tc(emb_table_T, W_comb):
    BR = 4096
    grid = ((VOCAB + BR - 1) // BR,)

    def body(t_ref, w_ref, o_ref):
        o_ref[...] = jax.lax.dot_general(
            t_ref[...], w_ref[:D_HALF, :],
            dimension_numbers=(((0,), (0,)), ((), ())),
            preferred_element_type=jnp.float32)

    return pl.pallas_call(
        body,
        grid=grid,
        in_specs=[
            pl.BlockSpec((D_HALF, BR), lambda i: (0, i)),
            pl.BlockSpec((D_MODEL, D_MODEL), lambda i: (0, 0)),
        ],
        out_specs=pl.BlockSpec((BR, D_MODEL), lambda i: (i, 0)),
        out_shape=jax.ShapeDtypeStruct((VOCAB, D_MODEL), jnp.float32),
    )(emb_table_T, W_comb)


def _gather_sc(table_t, ids_half):
    bpw = N_HALF // NW
    chunks = [(i * 256, 256) for i in range(12)] + [(3072, 128)]
    mesh = plsc.VectorSubcoreMesh(core_axis_name="c", subcore_axis_name="s")

    @functools.partial(
        pl.kernel,
        mesh=mesh,
        out_type=jax.ShapeDtypeStruct((N_HALF, D_MODEL), jnp.float32),
        scratch_types=[
            pltpu.VMEM((bpw,), jnp.int32),
            pltpu.VMEM((256, D_MODEL), jnp.float32),
            pltpu.VMEM((256, D_MODEL), jnp.float32),
            pltpu.SemaphoreType.DMA,
            pltpu.SemaphoreType.DMA,
            pltpu.SemaphoreType.DMA,
        ],
    )
    def k(table_hbm, ids_hbm, out_hbm, idx_v, buf0, buf1, gsem, osem0, osem1):
        wid = lax.axis_index("s") * 2 + lax.axis_index("c")
        base = wid * bpw
        pltpu.sync_copy(ids_hbm.at[pl.ds(base, bpw)], idx_v)
        bufs = (buf0, buf1)
        osems = (osem0, osem1)

        def issue(c, buf):
            off, ln = chunks[c]
            hs = []
            for j in range(0, ln, IDXW):
                hs.append(pltpu.async_copy(
                    table_hbm.at[idx_v.at[pl.ds(off + j, IDXW)]],
                    buf.at[pl.ds(j, IDXW)],
                    gsem))
            return hs

        pending = issue(0, bufs[0])
        out_h = [None, None]
        for c in range(len(chunks)):
            b = c & 1
            for h in pending:
                h.wait()
            if c + 1 < len(chunks):
                if out_h[1 - b] is not None:
                    out_h[1 - b].wait()
                pending = issue(c + 1, bufs[1 - b])
            off, ln = chunks[c]
            out_h[b] = pltpu.async_copy(
                bufs[b].at[pl.ds(0, ln)],
                out_hbm.at[pl.ds(base + off, ln)], osems[b])
        out_h[(len(chunks) - 1) & 1].wait()

    return k(table_t, ids_half)


def _dense_tc(gathered_h, stats_pack_h, W_big, b_big, W_bot, b_comb,
              half, prev_out=None):
    grid = (N_HALF // ROWS,)
    blk0 = half * (N_HALF // ROWS)

    def body(*refs):
        if half == 0:
            g_ref, sp_ref, wb_ref, bb_ref, wc_ref, bc_ref, out_ref = refs
        else:
            _, g_ref, sp_ref, wb_ref, bb_ref, wc_ref, bc_ref, out_ref = refs
        s = jnp.dot(sp_ref[...], wb_ref[...],
                    preferred_element_type=jnp.float32) + bb_ref[...]
        s = 0.5 * s * (1.0 + lax.erf(s * 0.7071067811865476))
        for g in range(TOK_PER_ROW):
            sl = pl.ds(g * PROWS, PROWS)
            bot = jnp.dot(s[:, g * D_HALF:(g + 1) * D_HALF], wc_ref[...],
                          preferred_element_type=jnp.float32)
            out_ref[sl, :] = g_ref[sl, :] + bot + bc_ref[...]

    in_specs = [
        pl.BlockSpec((ROWS, D_MODEL), lambda i: (i, 0)),
        pl.BlockSpec((PROWS, D_MODEL), lambda i: (i, 0)),
        pl.BlockSpec((D_MODEL, 8 * D_HALF), lambda i: (0, 0)),
        pl.BlockSpec((1, 8 * D_HALF), lambda i: (0, 0)),
        pl.BlockSpec((D_HALF, D_MODEL), lambda i: (0, 0)),
        pl.BlockSpec((1, D_MODEL), lambda i: (0, 0)),
    ]
    operands = [gathered_h, stats_pack_h, W_big,
                b_big.reshape(1, 8 * D_HALF), W_bot,
                b_comb.reshape(1, D_MODEL)]
    kwargs = {}
    if half == 1:
        in_specs = [pl.BlockSpec(memory_space=pl.ANY)] + in_specs
        operands = [prev_out] + operands
        kwargs["input_output_aliases"] = {0: 0}

    return pl.pallas_call(
        body,
        grid=grid,
        in_specs=in_specs,
        out_specs=pl.BlockSpec((ROWS, D_MODEL), lambda i: (blk0 + i, 0)),
        out_shape=jax.ShapeDtypeStruct((N_TOKENS, D_MODEL), jnp.float32),
        **kwargs,
    )(*operands)


def kernel(card_ids, card_stats, emb_table, W_stat, b_stat, W_comb, b_comb):
    B, L = card_ids.shape
    ids_flat = card_ids.T.reshape(N_TOKENS).astype(jnp.int32)
    table_t = _table_transform_tc(emb_table.T, W_comb)

    stats_lm = card_stats.transpose(1, 0, 2)
    stats_pad = jnp.pad(stats_lm, ((0, 0), (0, 0), (0, F_PAD - 10)))

    def pack(sp):
        return (sp.reshape(N_HALF // ROWS, TOK_PER_ROW, PROWS, F_PAD)
                .transpose(0, 2, 1, 3)
                .reshape(N_HALF // TOK_PER_ROW, 128))

    W16 = jnp.pad(W_stat, ((0, F_PAD - 10), (0, 0)))
    eye8 = jnp.eye(TOK_PER_ROW, dtype=jnp.float32)
    W_big = jnp.einsum("gh,fo->gfho", eye8, W16).reshape(128, 8 * D_HALF)
    b_big = jnp.tile(b_stat, TOK_PER_ROW)
    W_bot = W_comb[D_HALF:, :]

    g1 = _gather_sc(table_t, ids_flat[:N_HALF])
    g2 = _gather_sc(table_t, ids_flat[N_HALF:])
    sp1 = pack(stats_pad[:L // 2])
    sp2 = pack(stats_pad[L // 2:])
    o1 = _dense_tc(g1, sp1, W_big, b_big, W_bot, b_comb, half=0)
    out = _dense_tc(g2, sp2, W_big, b_big, W_bot, b_comb, half=1, prev_out=o1)
    return out.reshape(L, B, D_MODEL).transpose(1, 0, 2)


# speedup vs baseline: 2.6561x; 1.3796x over previous
import functools

import jax
import jax.numpy as jnp
from jax import lax
from jax.experimental import pallas as pl
from jax.experimental.pallas import tpu as pltpu
from jax.experimental.pallas import tpu_sc as plsc

VOCAB = 100000
D_HALF = 64
D_MODEL = 128
B_DIM = 4096
L_DIM = 50
N_TOKENS = B_DIM * L_DIM
BQ = B_DIM // 4
OCT_TOK = 8 * B_DIM

NW = 32
IDXW = 128

CHUNK_OCTS = (1, 2, 2, 2)
CHUNK_L0 = (0, 8, 24, 40)
CHUNK_TOKS = (32768, 65536, 65536, 40960)


def _table_transform_tc(emb_table_T, W_comb):
    BR = 4096
    grid = ((VOCAB + BR - 1) // BR,)

    def body(t_ref, w_ref, o_ref):
        o_ref[...] = jax.lax.dot_general(
            t_ref[...], w_ref[:D_HALF, :],
            dimension_numbers=(((0,), (0,)), ((), ())),
            preferred_element_type=jnp.float32)

    return pl.pallas_call(
        body,
        grid=grid,
        in_specs=[
            pl.BlockSpec((D_HALF, BR), lambda i: (0, i)),
            pl.BlockSpec((D_MODEL, D_MODEL), lambda i: (0, 0)),
        ],
        out_specs=pl.BlockSpec((BR, D_MODEL), lambda i: (i, 0)),
        out_shape=jax.ShapeDtypeStruct((VOCAB, D_MODEL), jnp.float32),
    )(emb_table_T, W_comb)


def _gather_sc(table_t, ids_chunk, n_tok):
    bpw = n_tok // NW
    nchunk = bpw // 256
    mesh = plsc.VectorSubcoreMesh(core_axis_name="c", subcore_axis_name="s")

    @functools.partial(
        pl.kernel,
        mesh=mesh,
        out_type=jax.ShapeDtypeStruct((n_tok, D_MODEL), jnp.float32),
        scratch_types=[
            pltpu.VMEM((bpw,), jnp.int32),
            pltpu.VMEM((256, D_MODEL), jnp.float32),
            pltpu.VMEM((256, D_MODEL), jnp.float32),
            pltpu.SemaphoreType.DMA,
            pltpu.SemaphoreType.DMA,
            pltpu.SemaphoreType.DMA,
        ],
    )
    def k(table_hbm, ids_hbm, out_hbm, idx_v, buf0, buf1, gsem, osem0, osem1):
        wid = lax.axis_index("s") * 2 + lax.axis_index("c")
        base = wid * bpw
        pltpu.sync_copy(ids_hbm.at[pl.ds(base, bpw)], idx_v)
        bufs = (buf0, buf1)
        osems = (osem0, osem1)

        def issue(c, buf):
            hs = []
            for j in range(2):
                hs.append(pltpu.async_copy(
                    table_hbm.at[idx_v.at[pl.ds(c * 256 + j * IDXW, IDXW)]],
                    buf.at[pl.ds(j * IDXW, IDXW)],
                    gsem))
            return hs

        pending = issue(0, bufs[0])
        out_h = [None, None]
        for c in range(nchunk):
            b = c & 1
            for h in pending:
                h.wait()
            if c + 1 < nchunk:
                if out_h[1 - b] is not None:
                    out_h[1 - b].wait()
                pending = issue(c + 1, bufs[1 - b])
            out_h[b] = pltpu.async_copy(
                bufs[b], out_hbm.at[pl.ds(base + c * 256, 256)], osems[b])
        out_h[(nchunk - 1) & 1].wait()

    return k(table_t, ids_chunk)


def _dense_tc(gathered_c, stats3, W_stat, b_stat, W_bot, b_comb,
              oct0, nocts, nl_real, prev_out=None):
    grid = (nocts, 4)
    g4 = gathered_c.reshape(nl_real, 4, BQ, D_MODEL)

    def body(*refs):
        if prev_out is None:
            g_ref, st_ref, ws_ref, bs_ref, wc_ref, bc_ref, out_ref = refs
        else:
            _, g_ref, st_ref, ws_ref, bs_ref, wc_ref, bc_ref, out_ref = refs
        for ll in range(8):
            x = st_ref[:, ll, :]
            s = jax.lax.dot_general(
                x, ws_ref[...],
                dimension_numbers=(((0,), (0,)), ((), ())),
                preferred_element_type=jnp.float32) + bs_ref[...]
            s = 0.5 * s * (1.0 + lax.erf(s * 0.7071067811865476))
            bot = jnp.dot(s, wc_ref[...], preferred_element_type=jnp.float32)
            out_ref[ll, 0] = g_ref[ll, 0] + bot + bc_ref[...]

    in_specs = [
        pl.BlockSpec((8, 1, BQ, D_MODEL), lambda li, bq: (li, bq, 0, 0)),
        pl.BlockSpec((10, 8, BQ), lambda li, bq: (0, oct0 + li, bq)),
        pl.BlockSpec((10, D_HALF), lambda li, bq: (0, 0)),
        pl.BlockSpec((1, D_HALF), lambda li, bq: (0, 0)),
        pl.BlockSpec((D_HALF, D_MODEL), lambda li, bq: (0, 0)),
        pl.BlockSpec((1, D_MODEL), lambda li, bq: (0, 0)),
    ]
    out_spec = pl.BlockSpec((8, 1, BQ, D_MODEL),
                            lambda li, bq: (oct0 + li, bq, 0, 0))
    operands = [g4, stats3, W_stat, b_stat.reshape(1, D_HALF),
                W_bot, b_comb.reshape(1, D_MODEL)]
    kwargs = {}
    if prev_out is not None:
        in_specs = [pl.BlockSpec(memory_space=pl.ANY)] + in_specs
        operands = [prev_out] + operands
        kwargs["input_output_aliases"] = {0: 0}

    return pl.pallas_call(
        body,
        grid=grid,
        in_specs=in_specs,
        out_specs=out_spec,
        out_shape=jax.ShapeDtypeStruct((L_DIM, 4, BQ, D_MODEL), jnp.float32),
        **kwargs,
    )(*operands)


def kernel(card_ids, card_stats, emb_table, W_stat, b_stat, W_comb, b_comb):
    B, L = card_ids.shape
    ids_flat = card_ids.T.reshape(N_TOKENS).astype(jnp.int32)
    stats3 = card_stats.transpose(2, 1, 0)
    W_bot = W_comb[D_HALF:, :]

    table_t = _table_transform_tc(emb_table.T, W_comb)

    gs = []
    off = 0
    for k in range(4):
        gs.append(_gather_sc(table_t, ids_flat[off:off + CHUNK_TOKS[k]],
                             CHUNK_TOKS[k]))
        off += CHUNK_TOKS[k]

    out = None
    for k in range(4):
        nl_real = CHUNK_TOKS[k] // B_DIM
        out = _dense_tc(gs[k], stats3, W_stat, b_stat, W_bot, b_comb,
                        CHUNK_L0[k] // 8, CHUNK_OCTS[k], nl_real,
                        prev_out=out)

    return out.reshape(L_DIM, B_DIM, D_MODEL).transpose(1, 0, 2)
